# baseline (speedup 1.0000x reference)
import jax
import jax.numpy as jnp
from jax import lax
from jax.experimental import pallas as pl
from jax.experimental.pallas import tpu as pltpu
from jax.experimental.pallas import tpu_sc as plsc

VOCAB = 1000000
EMBED = 32
B = 16384
C = 5

_INFO = plsc.get_sparse_core_info()
NC = _INFO.num_cores
NS = _INFO.num_subcores
NW = NC * NS
BPW = B // NW
PPW = BPW * C
CHUNK = 128
TCH = BPW // CHUNK
CCH = PPW // CHUNK
GRP = 16


def _sc_kernel(tgt_idx_hbm, ctx_idx_hbm, lbl_hbm, embt_hbm, embc_hbm,
               out_hbm,
               tgt_idx_v, ctx_idx_v, lbl_v, t_rows, c_rows, out_v, sem):
    wid = lax.axis_index("s") * NC + lax.axis_index("c")

    pltpu.sync_copy(tgt_idx_hbm.at[wid], tgt_idx_v)
    pltpu.sync_copy(ctx_idx_hbm.at[wid], ctx_idx_v)
    pltpu.sync_copy(lbl_hbm.at[pl.ds(wid * PPW, PPW)], lbl_v)

    dmas = []
    for j in range(TCH):
        dmas.append(pltpu.async_copy(
            embt_hbm.at[tgt_idx_v.at[j]],
            t_rows.at[pl.ds(j * CHUNK, CHUNK)], sem))
    for j in range(CCH):
        dmas.append(pltpu.async_copy(
            embc_hbm.at[ctx_idx_v.at[j]],
            c_rows.at[pl.ds(j * CHUNK, CHUNK)], sem))
    for dma in dmas:
        dma.wait()

    iota = lax.iota(jnp.int32, NS)

    def group_body(g, carry):
        b0 = g * GRP
        rows = b0 + iota
        pair0 = rows * C
        acc = [jnp.zeros((NS,), jnp.float32) for _ in range(C)]
        ctx_row = [pair0 + c for c in range(C)]
        for e in range(EMBED):
            col = jnp.full((NS,), e, jnp.int32)
            tv = plsc.load_gather(t_rows, [rows, col])
            for c in range(C):
                cv = plsc.load_gather(c_rows, [ctx_row[c], col])
                acc[c] = acc[c] + tv * cv
        for c in range(C):
            pos = pair0 + c
            lblv = plsc.load_gather(lbl_v, [pos])
            plsc.store_scatter(out_v, [pos], acc[c] * lblv)
        return carry

    lax.fori_loop(0, BPW // GRP, group_body, 0)

    pltpu.sync_copy(out_v, out_hbm.at[pl.ds(wid * PPW, PPW)])


def kernel(target, context, label, emb_target, emb_context):
    tgt2d = target.reshape(NW, TCH, CHUNK)
    ctx2d = context.reshape(NW, CCH, CHUNK)
    lblf = label.astype(jnp.float32).reshape(B * C)

    mesh = plsc.VectorSubcoreMesh(core_axis_name="c", subcore_axis_name="s")
    out = pl.kernel(
        _sc_kernel,
        mesh=mesh,
        compiler_params=pltpu.CompilerParams(needs_layout_passes=False,
                                             use_tc_tiling_on_sc=False),
        out_type=jax.ShapeDtypeStruct((B * C,), jnp.float32),
        scratch_types=[
            pltpu.VMEM((TCH, CHUNK), jnp.int32),
            pltpu.VMEM((CCH, CHUNK), jnp.int32),
            pltpu.VMEM((PPW,), jnp.float32),
            pltpu.VMEM((BPW, EMBED), jnp.float32),
            pltpu.VMEM((PPW, EMBED), jnp.float32),
            pltpu.VMEM((PPW,), jnp.float32),
            pltpu.SemaphoreType.DMA,
        ],
    )(tgt2d, ctx2d, lblf, emb_target, emb_context)
    return out.reshape(B, C)
